# baseline (speedup 1.0000x reference)
import functools

import jax
import jax.numpy as jnp
from jax import lax
from jax.experimental import pallas as pl
from jax.experimental.pallas import tpu as pltpu
from jax.experimental.pallas import tpu_sc as plsc

_TABLE_DIMS = (16, 16, 16, 16, 16, 8, 8, 8, 8, 8, 8)
_NUM_FIELDS = 11
_NC, _NS = 2, 16
_NW = _NC * _NS
_SPAD = 56


def _sc_body(B, S, GB, xq_hbm, *refs):
    w_hbm = refs[:_NUM_FIELDS]
    out_hbm = refs[_NUM_FIELDS : 2 * _NUM_FIELDS]
    scr = refs[2 * _NUM_FIELDS :]
    idx_v = scr[0]
    stages = scr[1 : 1 + _NUM_FIELDS]
    gsem = scr[1 + _NUM_FIELDS]
    ssem = scr[2 + _NUM_FIELDS]

    wid = lax.axis_index("s") * _NC + lax.axis_index("c")
    bpw = B // _NW
    b_lo = wid * bpw
    n_groups = bpw // GB

    def fire_gathers(p, c):
        def fire(k, c2):
            for i in range(_NUM_FIELDS):
                pltpu.async_copy(
                    w_hbm[i].at[idx_v.at[p, i, k]],
                    stages[i].at[p, k],
                    gsem,
                )
            return c2

        lax.fori_loop(0, GB, fire, c)

    def drain_gathers(p, c):
        def drain(k, c2):
            for i in range(_NUM_FIELDS):
                pltpu.make_async_copy(
                    w_hbm[i].at[idx_v.at[p, i, k]],
                    stages[i].at[p, k],
                    gsem,
                ).wait()
            return c2

        lax.fori_loop(0, GB, drain, c)

    def fire_scatters(p, b0):
        for i in range(_NUM_FIELDS):
            pltpu.async_copy(
                stages[i].at[p, :, pl.ds(0, S), :],
                out_hbm[i].at[pl.ds(b0, GB)],
                ssem,
            )

    def drain_scatters(p, b0):
        for i in range(_NUM_FIELDS):
            pltpu.make_async_copy(
                stages[i].at[p, :, pl.ds(0, S), :],
                out_hbm[i].at[pl.ds(b0, GB)],
                ssem,
            ).wait()

    def group(g, carry):
        p = lax.rem(g, 2)
        b0 = b_lo + g * GB
        pltpu.sync_copy(xq_hbm.at[:, pl.ds(b0, GB), :], idx_v.at[p])
        fire_gathers(p, 0)
        drain_gathers(p, 0)

        @pl.when(g >= 1)
        def _():
            drain_scatters(1 - p, b0 - GB)

        fire_scatters(p, b0)
        return carry

    lax.fori_loop(0, n_groups, group, 0)
    p_last = lax.rem(n_groups - 1, 2)
    drain_scatters(p_last, b_lo + (n_groups - 1) * GB)


def kernel(x, W0, W1, W2, W3, W4, W5, W6, W7, W8, W9, W10):
    Ws = (W0, W1, W2, W3, W4, W5, W6, W7, W8, W9, W10)
    B, S, F = x.shape
    GB = 8

    xq = jnp.pad(x.transpose(2, 0, 1), ((0, 0), (0, 0), (0, _SPAD - S)))

    out_type = tuple(
        jax.ShapeDtypeStruct((B, S, d), jnp.float32) for d in _TABLE_DIMS
    )
    scratch = [pltpu.VMEM((2, _NUM_FIELDS, GB, _SPAD), jnp.int32)]
    scratch += [
        pltpu.VMEM((2, GB, _SPAD, d), jnp.float32) for d in _TABLE_DIMS
    ]
    scratch += [pltpu.SemaphoreType.DMA, pltpu.SemaphoreType.DMA]

    mesh = plsc.VectorSubcoreMesh(core_axis_name="c", subcore_axis_name="s")
    fn = pl.kernel(
        functools.partial(_sc_body, B, S, GB),
        out_type=out_type,
        mesh=mesh,
        scratch_types=scratch,
        compiler_params=pltpu.CompilerParams(use_tc_tiling_on_sc=False),
    )
    return fn(xq, *Ws)
